# baseline (speedup 1.0000x reference)
import dataclasses
import functools

import jax
import jax.numpy as jnp
from jax import lax
from jax.experimental import pallas as pl
from jax.experimental.pallas import tpu as pltpu
from jax.experimental.pallas import tpu_sc as plsc

N = 10000
E = 320000
D = 128
R = 100

NC = 2
NS = 16
NW = NC * NS
EPT = E // NW
CH = 80
NCHUNK = EPT // CH
RB = 80
NRB = N // RB
HR = N // 16
GR = 128

_mesh = plsc.VectorSubcoreMesh(core_axis_name="c", subcore_axis_name="s")

_cp = pltpu.CompilerParams()
if "needs_layout_passes" in pltpu.CompilerParams.__dataclass_fields__:
    _cp = dataclasses.replace(_cp, needs_layout_passes=False)


@functools.partial(
    pl.kernel,
    mesh=_mesh,
    compiler_params=_cp,
    out_type=[
        jax.ShapeDtypeStruct((NW, GR * 128), jnp.float32),
        jax.ShapeDtypeStruct((NW, GR * 128), jnp.float32),
    ],
    scratch_types=[
        pltpu.VMEM((EPT,), jnp.int32),
        pltpu.VMEM((EPT,), jnp.int32),
        pltpu.VMEM((GR * 128,), jnp.float32),
        pltpu.VMEM((GR * 128,), jnp.float32),
    ],
)
def _deg_kernel(src_hbm, dst_hbm, hs_hbm, hd_hbm, sidx, didx, hist_s, hist_d):
    c = lax.axis_index("c")
    s = lax.axis_index("s")
    wid = c * NS + s

    pltpu.sync_copy(src_hbm.at[pl.ds(wid * EPT, EPT)], sidx)
    pltpu.sync_copy(dst_hbm.at[pl.ds(wid * EPT, EPT)], didx)

    zero = jnp.zeros((16,), jnp.float32)

    @pl.loop(0, GR * 8)
    def _(r):
        hist_s[pl.ds(r * 16, 16)] = zero
        hist_d[pl.ds(r * 16, 16)] = zero

    one = jnp.full((16,), 1.0, jnp.float32)

    def scram(n):
        r = lax.shift_right_logical(n, 4)
        t = n & 15
        return (lax.shift_left((r & 127), 7)
                + lax.shift_left(lax.shift_right_logical(r, 7), 4) + t)

    @pl.loop(0, EPT // 16)
    def _(i):
        sl = pl.ds(i * 16, 16)
        plsc.addupdate_scatter(hist_s, [scram(sidx[sl])], one)
        plsc.addupdate_scatter(hist_d, [scram(didx[sl])], one)

    pltpu.sync_copy(hist_s, hs_hbm.at[wid])
    pltpu.sync_copy(hist_d, hd_hbm.at[wid])


@functools.partial(
    pl.kernel,
    mesh=_mesh,
    out_type=jax.ShapeDtypeStruct((NC, N, D), jnp.float32),
    scratch_types=[
        pltpu.VMEM((CH,), jnp.int32),
        pltpu.VMEM((CH,), jnp.int32),
        pltpu.VMEM((CH,), jnp.int32),
        pltpu.VMEM((CH,), jnp.int32),
        pltpu.VMEM((CH,), jnp.int32),
        pltpu.VMEM((CH,), jnp.int32),
        pltpu.VMEM((CH, D), jnp.float32),
        pltpu.VMEM((CH, D), jnp.float32),
        pltpu.VMEM((CH, D), jnp.float32),
        pltpu.VMEM((CH, D), jnp.float32),
        pltpu.SemaphoreType.DMA,
        pltpu.SemaphoreType.DMA,
        pltpu.SemaphoreType.DMA,
        pltpu.SemaphoreType.DMA,
        pltpu.SemaphoreType.DMA,
        pltpu.SemaphoreType.DMA,
        pltpu.SemaphoreType.DMA,
        pltpu.SemaphoreType.DMA,
        pltpu.VMEM_SHARED((N, D), jnp.float32),
    ],
)
def _mix_kernel(feat_hbm, w_hbm, src_hbm, etype_hbm, dst_hbm, out_hbm,
                sidx0, sidx1, widx0, widx1, didx0, didx1,
                rows0, rows1, wrows0, wrows1,
                gf0, gw0, gf1, gw1, gi0, gi1, gd0, gd1, acc):
    c = lax.axis_index("c")
    s = lax.axis_index("s")
    wid = c * NS + s
    e0 = wid * EPT

    def gi_descs(k, sidx, widx, gi):
        return (pltpu.make_async_copy(src_hbm.at[pl.ds(e0 + k * CH, CH)],
                                      sidx, gi),
                pltpu.make_async_copy(etype_hbm.at[pl.ds(e0 + k * CH, CH)],
                                      widx, gi))

    def gd_desc(k, didx, gd):
        return pltpu.make_async_copy(dst_hbm.at[pl.ds(e0 + k * CH, CH)],
                                     didx, gd)

    def g_desc(rows, wrows, sidx, widx, gf, gw):
        return (pltpu.make_async_copy(feat_hbm.at[sidx], rows, gf),
                pltpu.make_async_copy(w_hbm.at[widx], wrows, gw))

    def start_gidx(k, sidx, widx, gi):
        c1, c2 = gi_descs(k, sidx, widx, gi)
        c1.start()
        c2.start()

    def consume_stage(k, kn, sidx, widx, didx, rows, wrows, gf, gw, gi, gd):
        cf, cw = g_desc(rows, wrows, sidx, widx, gf, gw)
        cf.wait()
        cw.wait()

        @pl.when(kn < NCHUNK)
        def _():
            start_gidx(kn, sidx, widx, gi)

        gd_desc(k, didx, gd).wait()

        @plsc.parallel_loop(0, CH, unroll=4)
        def _(r):
            for g in range(D // 16):
                sl = pl.ds(g * 16, 16)
                rows[r, sl] = rows[r, sl] * wrows[r, sl]

        pltpu.sync_copy(rows, acc.at[didx], add=True)

        @pl.when(kn < NCHUNK)
        def _():
            gd_desc(kn, didx, gd).start()
            c1, c2 = gi_descs(kn, sidx, widx, gi)
            c1.wait()
            c2.wait()
            cf2, cw2 = g_desc(rows, wrows, sidx, widx, gf, gw)
            cf2.start()
            cw2.start()

    _z = jnp.zeros((16,), jnp.float32)

    @pl.loop(0, CH)
    def _(r):
        for g in range(D // 16):
            rows0[r, pl.ds(g * 16, 16)] = _z

    @pl.loop(0, (NRB + NS - 1) // NS)
    def _(j):
        blk = s + j * NS

        @pl.when(blk < NRB)
        def _():
            pltpu.sync_copy(rows0, acc.at[pl.ds(blk * RB, RB)])

    plsc.subcore_barrier()

    start_gidx(0, sidx0, widx0, gi0)
    gd_desc(0, didx0, gd0).start()
    start_gidx(1, sidx1, widx1, gi1)
    gd_desc(1, didx1, gd1).start()
    c1, c2 = gi_descs(0, sidx0, widx0, gi0)
    c1.wait()
    c2.wait()
    cf, cw = g_desc(rows0, wrows0, sidx0, widx0, gf0, gw0)
    cf.start()
    cw.start()
    c1, c2 = gi_descs(1, sidx1, widx1, gi1)
    c1.wait()
    c2.wait()
    cf, cw = g_desc(rows1, wrows1, sidx1, widx1, gf1, gw1)
    cf.start()
    cw.start()

    @pl.loop(0, NCHUNK // 2)
    def _(j):
        a = 2 * j
        consume_stage(a, a + 2, sidx0, widx0, didx0, rows0, wrows0,
                      gf0, gw0, gi0, gd0)
        consume_stage(a + 1, a + 3, sidx1, widx1, didx1, rows1, wrows1,
                      gf1, gw1, gi1, gd1)

    consume_stage(NCHUNK - 1, NCHUNK + 1, sidx0, widx0, didx0,
                  rows0, wrows0, gf0, gw0, gi0, gd0)

    plsc.subcore_barrier()

    @pl.loop(0, (NRB + NS - 1) // NS)
    def _(j):
        blk = s + j * NS

        @pl.when(blk < NRB)
        def _():
            pltpu.sync_copy(acc.at[pl.ds(blk * RB, RB)],
                            out_hbm.at[c, pl.ds(blk * RB, RB)])


def _unscramble(deg2):
    return jnp.concatenate(
        [deg2[:, 16 * g:16 * (g + 1)] for g in range(5)], axis=0)[:HR]


def _prep_body(x_ref, hs_ref, hd_ref, fs_ref, sm_ref, nd_ref, ns_ref):
    deg_out = _unscramble(jnp.sum(hs_ref[...], axis=0))
    deg_in = _unscramble(jnp.sum(hd_ref[...], axis=0))
    ns = lax.rsqrt(jnp.maximum(deg_out, 1.0))
    cd = jnp.maximum(deg_in, 1.0)
    nd = lax.rsqrt(cd)
    ns_ref[...] = ns
    nd_ref[...] = nd
    sm_ref[...] = nd / cd
    fs_ref[...] = x_ref[...] * ns[:, :, None]


def _prep_call(x3, hs, hd):
    return pl.pallas_call(
        _prep_body,
        out_shape=[
            jax.ShapeDtypeStruct((HR, 16, D), jnp.float32),
            jax.ShapeDtypeStruct((HR, 16), jnp.float32),
            jax.ShapeDtypeStruct((HR, 16), jnp.float32),
            jax.ShapeDtypeStruct((HR, 16), jnp.float32),
        ],
    )(x3, hs, hd)


def _post_body(feat_ref, parts_ref, w_ref, sm_ref, nd_ref, ns_ref,
               h_ref, hs_ref):
    mix = parts_ref[0] + parts_ref[1]
    mm = lax.dot_general(feat_ref[...], w_ref[...],
                         (((2,), (0,)), ((), ())),
                         preferred_element_type=jnp.float32)
    rst = mm * nd_ref[...][:, :, None] + mix * sm_ref[...][:, :, None]
    h = jnp.maximum(rst, 0.0)
    h_ref[...] = h
    hs_ref[...] = h * ns_ref[...][:, :, None]


def _post_call(feat3, parts, loop_w, sm, nd, ns):
    return pl.pallas_call(
        _post_body,
        out_shape=[
            jax.ShapeDtypeStruct((HR, 16, D), jnp.float32),
            jax.ShapeDtypeStruct((HR, 16, D), jnp.float32),
        ],
    )(feat3, parts, loop_w, sm, nd, ns)


@jax.jit
def kernel(x, edge_index, etype, weight, loop_w1, loop_w2):
    src = edge_index[0]
    dst = edge_index[1]
    hs, hd = _deg_kernel(src, dst)
    x3 = x.reshape(HR, 16, D)
    fs3, sm, nd, ns = _prep_call(x3, hs.reshape(NW, GR, 128),
                                 hd.reshape(NW, GR, 128))
    parts1 = _mix_kernel(fs3.reshape(N, D), weight, src, etype, dst)
    h1, h1n = _post_call(x3, parts1.reshape(NC, HR, 16, D), loop_w1,
                         sm, nd, ns)
    parts2 = _mix_kernel(h1n.reshape(N, D), weight, src, etype, dst)
    h2, _ = _post_call(h1, parts2.reshape(NC, HR, 16, D), loop_w2,
                       sm, nd, ns)
    return h2.reshape(N, D)


# speedup vs baseline: 1.3453x; 1.3453x over previous
import dataclasses
import functools

import jax
import jax.numpy as jnp
from jax import lax
from jax.experimental import pallas as pl
from jax.experimental.pallas import tpu as pltpu
from jax.experimental.pallas import tpu_sc as plsc

N = 10000
E = 320000
D = 128
R = 100

NC = 2
NS = 16
NW = NC * NS
EPT = E // NW
CH = 80
NCHUNK = EPT // CH
RB = 80
NRB = N // RB
HR = N // 16
GR = 128

_mesh = plsc.VectorSubcoreMesh(core_axis_name="c", subcore_axis_name="s")

_cp = pltpu.CompilerParams()
if "needs_layout_passes" in pltpu.CompilerParams.__dataclass_fields__:
    _cp = dataclasses.replace(_cp, needs_layout_passes=False)


@functools.partial(
    pl.kernel,
    mesh=_mesh,
    compiler_params=_cp,
    out_type=[
        jax.ShapeDtypeStruct((NW, GR * 128), jnp.float32),
        jax.ShapeDtypeStruct((NW, GR * 128), jnp.float32),
    ],
    scratch_types=[
        pltpu.VMEM((EPT,), jnp.int32),
        pltpu.VMEM((EPT,), jnp.int32),
        pltpu.VMEM((GR * 128,), jnp.float32),
        pltpu.VMEM((GR * 128,), jnp.float32),
    ],
)
def _deg_kernel(src_hbm, dst_hbm, hs_hbm, hd_hbm, sidx, didx, hist_s, hist_d):
    c = lax.axis_index("c")
    s = lax.axis_index("s")
    wid = c * NS + s

    pltpu.sync_copy(src_hbm.at[pl.ds(wid * EPT, EPT)], sidx)
    pltpu.sync_copy(dst_hbm.at[pl.ds(wid * EPT, EPT)], didx)

    zero = jnp.zeros((16,), jnp.float32)

    @pl.loop(0, GR * 8)
    def _(r):
        hist_s[pl.ds(r * 16, 16)] = zero
        hist_d[pl.ds(r * 16, 16)] = zero

    one = jnp.full((16,), 1.0, jnp.float32)

    def scram(n):
        r = lax.shift_right_logical(n, 4)
        t = n & 15
        return (lax.shift_left((r & 127), 7)
                + lax.shift_left(lax.shift_right_logical(r, 7), 4) + t)

    @pl.loop(0, EPT // 16)
    def _(i):
        sl = pl.ds(i * 16, 16)
        plsc.addupdate_scatter(hist_s, [scram(sidx[sl])], one)
        plsc.addupdate_scatter(hist_d, [scram(didx[sl])], one)

    pltpu.sync_copy(hist_s, hs_hbm.at[wid])
    pltpu.sync_copy(hist_d, hd_hbm.at[wid])


@functools.partial(
    pl.kernel,
    mesh=_mesh,
    out_type=jax.ShapeDtypeStruct((NC, N, D), jnp.float32),
    scratch_types=[
        pltpu.VMEM((CH,), jnp.int32),
        pltpu.VMEM((CH,), jnp.int32),
        pltpu.VMEM((CH,), jnp.int32),
        pltpu.VMEM((CH,), jnp.int32),
        pltpu.VMEM((CH, D), jnp.float32),
        pltpu.VMEM((CH, D), jnp.float32),
        pltpu.VMEM((CH, D), jnp.float32),
        pltpu.VMEM((CH, D), jnp.float32),
        pltpu.SemaphoreType.DMA,
        pltpu.SemaphoreType.DMA,
        pltpu.SemaphoreType.DMA,
        pltpu.SemaphoreType.DMA,
        pltpu.SemaphoreType.DMA,
        pltpu.SemaphoreType.DMA,
        pltpu.SemaphoreType.DMA,
        pltpu.SemaphoreType.DMA,
        pltpu.VMEM_SHARED((N, D), jnp.float32),
    ],
)
def _mix_kernel(feat_hbm, wg_hbm, src_hbm, dst_hbm, out_hbm,
                sidx0, sidx1, didx0, didx1,
                rows0, rows1, wrows0, wrows1,
                gf0, gw0, gf1, gw1, gi0, gi1, gd0, gd1, acc):
    c = lax.axis_index("c")
    s = lax.axis_index("s")
    wid = c * NS + s
    e0 = wid * EPT

    def gi_desc(k, sidx, gi):
        return pltpu.make_async_copy(src_hbm.at[pl.ds(e0 + k * CH, CH)],
                                     sidx, gi)

    def gd_desc(k, didx, gd):
        return pltpu.make_async_copy(dst_hbm.at[pl.ds(e0 + k * CH, CH)],
                                     didx, gd)

    def g_descs(k, rows, wrows, sidx, gf, gw):
        return (pltpu.make_async_copy(feat_hbm.at[sidx], rows, gf),
                pltpu.make_async_copy(wg_hbm.at[pl.ds(e0 + k * CH, CH)],
                                      wrows, gw))

    def consume_stage(k, kn, sidx, didx, rows, wrows, gf, gw, gi, gd):
        cf, cw = g_descs(k, rows, wrows, sidx, gf, gw)
        cf.wait()
        cw.wait()

        @pl.when(kn < NCHUNK)
        def _():
            gi_desc(kn, sidx, gi).start()

        gd_desc(k, didx, gd).wait()

        @plsc.parallel_loop(0, CH, unroll=4)
        def _(r):
            for g in range(D // 16):
                sl = pl.ds(g * 16, 16)
                rows[r, sl] = rows[r, sl] * wrows[r, sl]

        pltpu.sync_copy(rows, acc.at[didx], add=True)

        @pl.when(kn < NCHUNK)
        def _():
            gd_desc(kn, didx, gd).start()
            gi_desc(kn, sidx, gi).wait()
            cf2, cw2 = g_descs(kn, rows, wrows, sidx, gf, gw)
            cf2.start()
            cw2.start()

    _z = jnp.zeros((16,), jnp.float32)

    @pl.loop(0, CH)
    def _(r):
        for g in range(D // 16):
            rows0[r, pl.ds(g * 16, 16)] = _z

    @pl.loop(0, (NRB + NS - 1) // NS)
    def _(j):
        blk = s + j * NS

        @pl.when(blk < NRB)
        def _():
            pltpu.sync_copy(rows0, acc.at[pl.ds(blk * RB, RB)])

    plsc.subcore_barrier()

    gi_desc(0, sidx0, gi0).start()
    gd_desc(0, didx0, gd0).start()
    gi_desc(1, sidx1, gi1).start()
    gd_desc(1, didx1, gd1).start()
    gi_desc(0, sidx0, gi0).wait()
    cf, cw = g_descs(0, rows0, wrows0, sidx0, gf0, gw0)
    cf.start()
    cw.start()
    gi_desc(1, sidx1, gi1).wait()
    cf, cw = g_descs(1, rows1, wrows1, sidx1, gf1, gw1)
    cf.start()
    cw.start()

    @pl.loop(0, NCHUNK // 2)
    def _(j):
        a = 2 * j
        consume_stage(a, a + 2, sidx0, didx0, rows0, wrows0,
                      gf0, gw0, gi0, gd0)
        consume_stage(a + 1, a + 3, sidx1, didx1, rows1, wrows1,
                      gf1, gw1, gi1, gd1)

    consume_stage(NCHUNK - 1, NCHUNK + 1, sidx0, didx0,
                  rows0, wrows0, gf0, gw0, gi0, gd0)

    plsc.subcore_barrier()

    @pl.loop(0, (NRB + NS - 1) // NS)
    def _(j):
        blk = s + j * NS

        @pl.when(blk < NRB)
        def _():
            pltpu.sync_copy(acc.at[pl.ds(blk * RB, RB)],
                            out_hbm.at[c, pl.ds(blk * RB, RB)])


BSE = 2000
NBE = E // BSE


def _wg_body(et_ref, w_ref, wg_ref):
    oh = (et_ref[0] == lax.broadcasted_iota(jnp.int32, (R, BSE), 0))
    wg_ref[...] = lax.dot_general(oh.astype(jnp.float32), w_ref[...],
                                  (((0,), (0,)), ((), ())),
                                  preferred_element_type=jnp.float32)


def _wg_call(et3, weight):
    return pl.pallas_call(
        _wg_body,
        grid=(NBE,),
        in_specs=[
            pl.BlockSpec((1, 1, BSE), lambda i: (i, 0, 0)),
            pl.BlockSpec((R, D), lambda i: (0, 0)),
        ],
        out_specs=pl.BlockSpec((BSE, D), lambda i: (i, 0)),
        out_shape=jax.ShapeDtypeStruct((E, D), jnp.float32),
    )(et3, weight)

def _unscramble(deg2):
    return jnp.concatenate(
        [deg2[:, 16 * g:16 * (g + 1)] for g in range(5)], axis=0)[:HR]


def _prep_body(x_ref, hs_ref, hd_ref, fs_ref, sm_ref, nd_ref, ns_ref):
    deg_out = _unscramble(jnp.sum(hs_ref[...], axis=0))
    deg_in = _unscramble(jnp.sum(hd_ref[...], axis=0))
    ns = lax.rsqrt(jnp.maximum(deg_out, 1.0))
    cd = jnp.maximum(deg_in, 1.0)
    nd = lax.rsqrt(cd)
    ns_ref[...] = ns
    nd_ref[...] = nd
    sm_ref[...] = nd / cd
    fs_ref[...] = x_ref[...] * ns[:, :, None]


def _prep_call(x3, hs, hd):
    return pl.pallas_call(
        _prep_body,
        out_shape=[
            jax.ShapeDtypeStruct((HR, 16, D), jnp.float32),
            jax.ShapeDtypeStruct((HR, 16), jnp.float32),
            jax.ShapeDtypeStruct((HR, 16), jnp.float32),
            jax.ShapeDtypeStruct((HR, 16), jnp.float32),
        ],
    )(x3, hs, hd)


def _post_body(feat_ref, parts_ref, w_ref, sm_ref, nd_ref, ns_ref,
               h_ref, hs_ref):
    mix = parts_ref[0] + parts_ref[1]
    mm = lax.dot_general(feat_ref[...], w_ref[...],
                         (((2,), (0,)), ((), ())),
                         preferred_element_type=jnp.float32)
    rst = mm * nd_ref[...][:, :, None] + mix * sm_ref[...][:, :, None]
    h = jnp.maximum(rst, 0.0)
    h_ref[...] = h
    hs_ref[...] = h * ns_ref[...][:, :, None]


def _post_call(feat3, parts, loop_w, sm, nd, ns):
    return pl.pallas_call(
        _post_body,
        out_shape=[
            jax.ShapeDtypeStruct((HR, 16, D), jnp.float32),
            jax.ShapeDtypeStruct((HR, 16, D), jnp.float32),
        ],
    )(feat3, parts, loop_w, sm, nd, ns)


@jax.jit
def kernel(x, edge_index, etype, weight, loop_w1, loop_w2):
    src = edge_index[0]
    dst = edge_index[1]
    hs, hd = _deg_kernel(src, dst)
    wg = _wg_call(etype.reshape(NBE, 1, BSE), weight)
    x3 = x.reshape(HR, 16, D)
    fs3, sm, nd, ns = _prep_call(x3, hs.reshape(NW, GR, 128),
                                 hd.reshape(NW, GR, 128))
    parts1 = _mix_kernel(fs3.reshape(N, D), wg, src, dst)
    h1, h1n = _post_call(x3, parts1.reshape(NC, HR, 16, D), loop_w1,
                         sm, nd, ns)
    parts2 = _mix_kernel(h1n.reshape(N, D), wg, src, dst)
    h2, _ = _post_call(h1, parts2.reshape(NC, HR, 16, D), loop_w2,
                       sm, nd, ns)
    return h2.reshape(N, D)
